# baseline (speedup 1.0000x reference)
import jax
import jax.numpy as jnp
from jax import lax
from jax.experimental import pallas as pl
from jax.experimental.pallas import tpu as pltpu
from jax.experimental.pallas import tpu_sc as plsc

B, P, L = 16384, 128, 100
NC, NS, LANES = 2, 16, 16
NW = NC * NS
BPW = B // NW
GROUPS = BPW // LANES
UNROLL = 16
Q = 4


def _sc_body(logits_hbm, label_hbm, tbl_hbm, out_hbm, x_v, lbl_v, tbl_v, tot_v,
             sem0, sem1, sem2, sem3, sem_l, sem_t):
    c = lax.axis_index("c")
    s = lax.axis_index("s")
    wid = c * NS + s
    base = wid * BPW
    chunk = BPW * P // Q

    sems = [sem0, sem1, sem2, sem3]
    cps = [
        pltpu.async_copy(
            logits_hbm.at[pl.ds(base * P + q * chunk, chunk)],
            x_v.at[pl.ds(q * chunk, chunk)], sems[q])
        for q in range(Q)
    ]
    cl = pltpu.async_copy(label_hbm.at[pl.ds(base, BPW)], lbl_v, sem_l)
    ct = pltpu.async_copy(tbl_hbm, tbl_v, sem_t)

    lane = lax.iota(jnp.int32, LANES)
    zero = jnp.zeros((LANES,), jnp.float32)

    def group_body(g, tot):
        lbl = lbl_v[pl.ds(g * LANES, LANES)]
        xb = g * (LANES * P) + lane * (P + 1)
        mb = lbl * P + lane

        def step(_, carry):
            a0, a1, a2, a3, ix, im = carry
            accs = [a0, a1, a2, a3]
            for u in range(UNROLL):
                xv = plsc.load_gather(x_v, [ix + u])
                mv = plsc.load_gather(tbl_v, [im + u])
                d = xv - mv
                accs[u % 4] = accs[u % 4] + d * d
            return (accs[0], accs[1], accs[2], accs[3],
                    ix + UNROLL, im + UNROLL)

        a0, a1, a2, a3, ix, im = lax.fori_loop(
            0, (P - LANES) // UNROLL, step, (zero, zero, zero, zero, xb, mb))

        xrow = g * (LANES * P) + lane * P
        for u in range(LANES):
            fu = (lane + (P - LANES) + u) & (P - 1)
            xv = plsc.load_gather(x_v, [xrow + fu])
            mv = plsc.load_gather(tbl_v, [lbl * P + fu])
            d = xv - mv
            accs = [a0, a1, a2, a3]
            accs[u % 4] = accs[u % 4] + d * d
            a0, a1, a2, a3 = accs

        ss = (a0 + a1) + (a2 + a3)

        xc = jnp.maximum(ss, jnp.float32(1e-30))
        yi = jnp.int32(0x5F3759DF) - lax.shift_right_logical(
            lax.bitcast_convert_type(xc, jnp.int32), 1)
        y = lax.bitcast_convert_type(yi, jnp.float32)
        for _ in range(3):
            y = y * (jnp.float32(1.5) - jnp.float32(0.5) * xc * y * y)
        return tot + xc * y

    cl.wait()
    ct.wait()
    tot = zero
    gq = GROUPS // Q
    for q in range(Q):
        cps[q].wait()
        tot = lax.fori_loop(q * gq, (q + 1) * gq, group_body, tot)
    tot_v[...] = tot
    pltpu.sync_copy(tot_v, out_hbm.at[wid])


def _tc_finish_body(x_ref, o_ref):
    o_ref[0, 0] = jnp.sum(x_ref[...]) * (1.0 / B)


@jax.jit
def kernel(logits, label, mean_expand):
    label = label.astype(jnp.int32)
    sc = pl.kernel(
        _sc_body,
        out_type=jax.ShapeDtypeStruct((NW, LANES), jnp.float32),
        mesh=plsc.VectorSubcoreMesh(core_axis_name="c", subcore_axis_name="s"),
        compiler_params=pltpu.CompilerParams(needs_layout_passes=False),
        scratch_types=[
            pltpu.VMEM((BPW * P,), jnp.float32),
            pltpu.VMEM((BPW,), jnp.int32),
            pltpu.VMEM((L * P,), jnp.float32),
            pltpu.VMEM((LANES,), jnp.float32),
            pltpu.SemaphoreType.DMA,
            pltpu.SemaphoreType.DMA,
            pltpu.SemaphoreType.DMA,
            pltpu.SemaphoreType.DMA,
            pltpu.SemaphoreType.DMA,
            pltpu.SemaphoreType.DMA,
        ],
    )
    partials = sc(logits.reshape(B * P), label, mean_expand.reshape(L * P))

    loss = pl.pallas_call(
        _tc_finish_body,
        out_shape=jax.ShapeDtypeStruct((1, 1), jnp.float32),
        out_specs=pl.BlockSpec(memory_space=pltpu.SMEM),
    )(partials.reshape(4, 128))
    return loss[0, 0]


# speedup vs baseline: 1.0139x; 1.0139x over previous
import jax
import jax.numpy as jnp
from jax import lax
from jax.experimental import pallas as pl
from jax.experimental.pallas import tpu as pltpu
from jax.experimental.pallas import tpu_sc as plsc

B, P, L = 16384, 128, 100
NC, NS, LANES = 2, 16, 16
NW = NC * NS
BPW = B // NW
GROUPS = BPW // LANES
UNROLL = 8
Q = 4


def _sc_body(logits_hbm, label_hbm, tbl_hbm, out_hbm, x_v, lbl_v, tbl_v, tot_v,
             sem0, sem1, sem2, sem3, sem_l, sem_t):
    c = lax.axis_index("c")
    s = lax.axis_index("s")
    wid = c * NS + s
    base = wid * BPW
    chunk = BPW * P // Q

    sems = [sem0, sem1, sem2, sem3]
    cps = [
        pltpu.async_copy(
            logits_hbm.at[pl.ds(base * P + q * chunk, chunk)],
            x_v.at[pl.ds(q * chunk, chunk)], sems[q])
        for q in range(Q)
    ]
    cl = pltpu.async_copy(label_hbm.at[pl.ds(base, BPW)], lbl_v, sem_l)
    ct = pltpu.async_copy(tbl_hbm, tbl_v, sem_t)

    lane = lax.iota(jnp.int32, LANES)
    zero = jnp.zeros((LANES,), jnp.float32)

    def group_body(g, tot):
        lbl = lbl_v[pl.ds(g * LANES, LANES)]
        xb = g * (LANES * P) + lane * (P + 1)
        mb = lbl * P + lane

        def step(_, carry):
            a0, a1, a2, a3, ix, im = carry
            accs = [a0, a1, a2, a3]
            for u in range(UNROLL):
                xv = plsc.load_gather(x_v, [ix + u])
                mv = plsc.load_gather(tbl_v, [im + u])
                d = xv - mv
                accs[u % 4] = accs[u % 4] + d * d
            return (accs[0], accs[1], accs[2], accs[3],
                    ix + UNROLL, im + UNROLL)

        a0, a1, a2, a3, ix, im = lax.fori_loop(
            0, (P - LANES) // UNROLL, step, (zero, zero, zero, zero, xb, mb))

        xrow = g * (LANES * P) + lane * P
        for u in range(LANES):
            fu = (lane + (P - LANES) + u) & (P - 1)
            xv = plsc.load_gather(x_v, [xrow + fu])
            mv = plsc.load_gather(tbl_v, [lbl * P + fu])
            d = xv - mv
            accs = [a0, a1, a2, a3]
            accs[u % 4] = accs[u % 4] + d * d
            a0, a1, a2, a3 = accs

        ss = (a0 + a1) + (a2 + a3)

        xc = jnp.maximum(ss, jnp.float32(1e-30))
        yi = jnp.int32(0x5F3759DF) - lax.shift_right_logical(
            lax.bitcast_convert_type(xc, jnp.int32), 1)
        y = lax.bitcast_convert_type(yi, jnp.float32)
        for _ in range(3):
            y = y * (jnp.float32(1.5) - jnp.float32(0.5) * xc * y * y)
        return tot + xc * y

    cl.wait()
    ct.wait()
    tot = zero
    gq = GROUPS // Q
    for q in range(Q):
        cps[q].wait()
        tot = lax.fori_loop(q * gq, (q + 1) * gq, group_body, tot)
    tot_v[...] = tot
    pltpu.sync_copy(tot_v, out_hbm.at[wid])


def _tc_finish_body(x_ref, o_ref):
    o_ref[0, 0] = jnp.sum(x_ref[...]) * (1.0 / B)


@jax.jit
def kernel(logits, label, mean_expand):
    label = label.astype(jnp.int32)
    sc = pl.kernel(
        _sc_body,
        out_type=jax.ShapeDtypeStruct((NW, LANES), jnp.float32),
        mesh=plsc.VectorSubcoreMesh(core_axis_name="c", subcore_axis_name="s"),
        compiler_params=pltpu.CompilerParams(needs_layout_passes=False),
        scratch_types=[
            pltpu.VMEM((BPW * P,), jnp.float32),
            pltpu.VMEM((BPW,), jnp.int32),
            pltpu.VMEM((L * P,), jnp.float32),
            pltpu.VMEM((LANES,), jnp.float32),
            pltpu.SemaphoreType.DMA,
            pltpu.SemaphoreType.DMA,
            pltpu.SemaphoreType.DMA,
            pltpu.SemaphoreType.DMA,
            pltpu.SemaphoreType.DMA,
            pltpu.SemaphoreType.DMA,
        ],
    )
    partials = sc(logits.reshape(B * P), label, mean_expand.reshape(L * P))

    loss = pl.pallas_call(
        _tc_finish_body,
        out_shape=jax.ShapeDtypeStruct((1, 1), jnp.float32),
        out_specs=pl.BlockSpec(memory_space=pltpu.SMEM),
    )(partials.reshape(4, 128))
    return loss[0, 0]
